# baseline (speedup 1.0000x reference)
import jax
import jax.numpy as jnp
from jax import lax
from jax.experimental import pallas as pl
from jax.experimental.pallas import tpu as pltpu
from jax.experimental.pallas import tpu_sc as plsc

VOCAB = 100000
EMBED_DIM = 64
BATCH = 4096
SEQ = 200

NC = 2
NS = 16
NW = NC * NS
SEQ_PER_W = BATCH // NW
LANES = 16
VPR = EMBED_DIM // LANES
NBUF = 4
PD = 2
HSEQ = SEQ // 2
L0 = 104
L1 = SEQ - L0
UNITS = 2 * SEQ_PER_W


def _body(x_ref, tok_ref, pos_ref, out_ref, idx_v, pos_v,
          rows0, rows1, rows2, rows3, pk0, pk1, pk2, pk3,
          g0, g1, g2, g3, s0, s1, s2, s3):
    bufs = (rows0, rows1, rows2, rows3)
    pks = (pk0, pk1, pk2, pk3)
    gsems = (g0, g1, g2, g3)
    ssems = (s0, s1, s2, s3)

    wid = lax.axis_index("s") * NC + lax.axis_index("c")
    base_seq = wid * SEQ_PER_W

    pltpu.sync_copy(x_ref.at[pl.ds(base_seq, SEQ_PER_W)], idx_v)
    pltpu.sync_copy(pos_ref.at[pl.ds(0, SEQ)], pos_v)

    def issue_gather(seq_local, h, rows, gsem):
        ln = L0 if h == 0 else L1
        pltpu.async_copy(tok_ref.at[idx_v.at[seq_local, pl.ds(h * L0, ln)]],
                         rows.at[pl.ds(0, ln)], gsem)

    def drain_gather(seq_local, h, rows, gsem):
        ln = L0 if h == 0 else L1
        pltpu.make_async_copy(tok_ref.at[idx_v.at[seq_local, pl.ds(h * L0, ln)]],
                              rows.at[pl.ds(0, ln)], gsem).wait()

    def drain_store(h, pk, ssem):
        hn = (L0 if h == 0 else L1) // 2
        pltpu.make_async_copy(pk.at[pl.ds(0, hn)],
                              out_ref.at[pl.ds(0, hn)], ssem).wait()

    for b in range(PD):
        issue_gather(0, b % 2, bufs[b], gsems[b])

    def quad(t, carry):
        u4 = t * NBUF
        for b in range(NBUF):
            u = u4 + b
            h = b % 2
            ln = L0 if h == 0 else L1
            hn = ln // 2
            seq_local = 2 * t + b // 2
            rows, pk, gsem, ssem = bufs[b], pks[b], gsems[b], ssems[b]
            drain_gather(seq_local, h, rows, gsem)

            j = (b + PD) % NBUF

            @pl.when(u + PD < UNITS)
            def _():
                @pl.when(u + PD >= NBUF)
                def _():
                    drain_store(h, pks[j], ssems[j])
                issue_gather(seq_local + 1, h, bufs[j], gsems[j])

            @plsc.parallel_loop(0, hn)
            def _(rr):
                for hh in range(2):
                    r = 2 * rr + hh
                    for d in range(VPR):
                        src = pl.ds(d * LANES, LANES)
                        dst = pl.ds(hh * EMBED_DIM + d * LANES, LANES)
                        pk[rr, dst] = rows[r, src] + pos_v[h * L0 + r, src]

            out_row = (base_seq + seq_local) * HSEQ + h * (L0 // 2)
            pltpu.async_copy(pk.at[pl.ds(0, hn)],
                             out_ref.at[pl.ds(out_row, hn)], ssem)
        return carry

    lax.fori_loop(0, UNITS // NBUF, quad, 0)

    for b in range(NBUF):
        drain_store(b % 2, pks[b], ssems[b])


def kernel(x, token_table, pos_table):
    x32 = x.astype(jnp.int32)
    mesh = plsc.VectorSubcoreMesh(core_axis_name="c", subcore_axis_name="s")
    k = pl.kernel(
        _body,
        mesh=mesh,
        compiler_params=pltpu.CompilerParams(use_tc_tiling_on_sc=False),
        out_type=jax.ShapeDtypeStruct((BATCH * HSEQ, 2 * EMBED_DIM),
                                      jnp.float32),
        scratch_types=[
            pltpu.VMEM((SEQ_PER_W, SEQ), jnp.int32),
            pltpu.VMEM((SEQ, EMBED_DIM), jnp.float32),
        ] + [pltpu.VMEM((L0, EMBED_DIM), jnp.float32) for _ in range(NBUF)]
          + [pltpu.VMEM((L0 // 2, 2 * EMBED_DIM), jnp.float32)
             for _ in range(NBUF)]
          + [pltpu.SemaphoreType.DMA for _ in range(2 * NBUF)],
    )
    out_packed = k(x32, token_table, pos_table)
    return out_packed.reshape(BATCH, SEQ, EMBED_DIM)
